# Initial kernel scaffold; baseline (speedup 1.0000x reference)
import jax
import jax.numpy as jnp
from jax.experimental import pallas as pl


def kernel(out, y):
    raise NotImplementedError("write your pallas kernel here")


# speedup vs baseline: 1.0004x; 1.0004x over previous
import jax
import jax.numpy as jnp
from jax.experimental import pallas as pl
from jax.experimental.pallas import tpu as pltpu

ROWS = 128
COLS = 32768
K = 64
CHUNK = 4096


def _body(out_ref, y_ref, res_ref, acc_ref):
    pid = pl.program_id(0)
    nsteps = pl.num_programs(0)

    d = out_ref[...] - y_ref[...]
    partial = jnp.sum(d * d, axis=1, keepdims=True)

    @pl.when(pid == 0)
    def _init():
        acc_ref[...] = partial

    @pl.when(pid != 0)
    def _accum():
        acc_ref[...] += partial

    @pl.when(pid == nsteps - 1)
    def _finish():
        v = acc_ref[...].reshape(1, ROWS) * (1.0 / COLS)
        gt = v > v.reshape(ROWS, 1)
        rank = jnp.sum(gt.astype(jnp.float32), axis=1).reshape(1, ROWS)
        cand = rank < K
        t = jnp.min(jnp.where(cand, v, jnp.inf))
        above = v > t
        n_above = jnp.sum(above.astype(jnp.float32))
        total = jnp.sum(jnp.where(above, v, 0.0)) + t * (K - n_above)
        res_ref[...] = total.reshape(1, 1)


def kernel(out, y):
    nsteps = COLS // CHUNK
    res = pl.pallas_call(
        _body,
        grid=(nsteps,),
        in_specs=[
            pl.BlockSpec((ROWS, CHUNK), lambda i: (0, i)),
            pl.BlockSpec((ROWS, CHUNK), lambda i: (0, i)),
        ],
        out_specs=pl.BlockSpec((1, 1), lambda i: (0, 0)),
        out_shape=jax.ShapeDtypeStruct((1, 1), jnp.float32),
        scratch_shapes=[pltpu.VMEM((ROWS, 1), jnp.float32)],
        compiler_params=pltpu.CompilerParams(
            dimension_semantics=("arbitrary",),
        ),
    )(out, y)
    return res[0, 0]
